# baseline (device time: 84593 ns/iter reference)
import jax
import jax.numpy as jnp
from jax import lax
from jax.experimental import pallas as pl
from jax.experimental.pallas import tpu as pltpu

N_DEV = 4


def kernel(x, router_W, route_idx, expert_W, shared_W):
    n_tok, d = x.shape
    n_exp = router_W.shape[1]
    e_loc, _, h = expert_W.shape

    def body(x_ref, rw_ref, idx_ref, ew_ref, sw_ref, out_ref,
             comm_ref, send_sems, recv_sems):
        my = lax.axis_index("i")
        left = lax.rem(my + N_DEV - 1, N_DEV)
        right = lax.rem(my + 1, N_DEV)

        barrier_sem = pltpu.get_barrier_semaphore()
        for nbr in (left, right):
            pl.semaphore_signal(
                barrier_sem, inc=1,
                device_id=(nbr,), device_id_type=pl.DeviceIdType.MESH,
            )
        pl.semaphore_wait(barrier_sem, 2)

        xv = x_ref[:, :]
        idx = idx_ref[:, :]

        scores = jnp.dot(xv, rw_ref[:, :], preferred_element_type=jnp.float32)
        s_max = jnp.max(scores, axis=-1, keepdims=True)
        p = jnp.exp(scores - s_max)
        probs = p / jnp.sum(p, axis=-1, keepdims=True)
        onehot = (idx == lax.broadcasted_iota(jnp.int32, (n_tok, n_exp), 1))
        gate = jnp.sum(probs * onehot.astype(jnp.float32), axis=-1,
                       keepdims=True)

        def accum_chunk(acc, w_ref, origin):
            base = origin * e_loc
            for j in range(e_loc):
                coeff = jnp.where(idx == base + j, gate, 0.0)
                acc = acc + jnp.dot(xv * coeff, w_ref[j, :, :],
                                    preferred_element_type=jnp.float32)
            return acc

        rdma0 = pltpu.make_async_remote_copy(
            src_ref=ew_ref,
            dst_ref=comm_ref.at[0],
            send_sem=send_sems.at[0],
            recv_sem=recv_sems.at[0],
            device_id=(right,),
            device_id_type=pl.DeviceIdType.MESH,
        )
        rdma0.start()
        acc = jnp.dot(xv, sw_ref[:, :], preferred_element_type=jnp.float32)
        acc = accum_chunk(acc, ew_ref, my)
        rdma0.wait()

        for hop in range(1, N_DEV - 1):
            rd = pltpu.make_async_remote_copy(
                src_ref=comm_ref.at[hop - 1],
                dst_ref=comm_ref.at[hop],
                send_sem=send_sems.at[hop],
                recv_sem=recv_sems.at[hop],
                device_id=(right,),
                device_id_type=pl.DeviceIdType.MESH,
            )
            rd.start()
            acc = accum_chunk(acc, comm_ref.at[hop - 1],
                              lax.rem(my + N_DEV - hop, N_DEV))
            rd.wait()

        acc = accum_chunk(acc, comm_ref.at[N_DEV - 2],
                          lax.rem(my + 1, N_DEV))
        out_ref[:, :] = acc

    return pl.pallas_call(
        body,
        out_shape=jax.ShapeDtypeStruct((n_tok, h), jnp.float32),
        in_specs=[pl.BlockSpec(memory_space=pltpu.VMEM)] * 5,
        out_specs=pl.BlockSpec(memory_space=pltpu.VMEM),
        scratch_shapes=[
            pltpu.VMEM((N_DEV - 1, e_loc, d, h), jnp.float32),
            pltpu.SemaphoreType.DMA((N_DEV - 1,)),
            pltpu.SemaphoreType.DMA((N_DEV - 1,)),
        ],
        compiler_params=pltpu.CompilerParams(collective_id=0),
    )(x, router_W, route_idx, expert_W, shared_W)


# device time: 51134 ns/iter; 1.6543x vs baseline; 1.6543x over previous
import jax
import jax.numpy as jnp
from jax import lax
from jax.experimental import pallas as pl
from jax.experimental.pallas import tpu as pltpu

N_DEV = 4


def kernel(x, router_W, route_idx, expert_W, shared_W):
    n_tok, d = x.shape
    n_exp = router_W.shape[1]
    e_loc, _, h = expert_W.shape

    def body(x_ref, rw_ref, idx_ref, ew_ref, sw_ref, out_ref,
             comm_ref, ew_bf_ref, send_sems, recv_sems):
        my = lax.axis_index("i")
        left = lax.rem(my + N_DEV - 1, N_DEV)
        right = lax.rem(my + 1, N_DEV)

        barrier_sem = pltpu.get_barrier_semaphore()
        for nbr in (left, right):
            pl.semaphore_signal(
                barrier_sem, inc=1,
                device_id=(nbr,), device_id_type=pl.DeviceIdType.MESH,
            )
        pl.semaphore_wait(barrier_sem, 2)

        xv = x_ref[:, :]
        idx = idx_ref[:, :]

        ew_bf_ref[:, :, :] = ew_ref[:, :, :].astype(jnp.bfloat16)
        xv_bf = xv.astype(jnp.bfloat16)

        scores = jnp.dot(xv, rw_ref[:, :], preferred_element_type=jnp.float32)
        s_max = jnp.max(scores, axis=-1, keepdims=True)
        p = jnp.exp(scores - s_max)
        probs = p / jnp.sum(p, axis=-1, keepdims=True)
        onehot = (idx == lax.broadcasted_iota(jnp.int32, (n_tok, n_exp), 1))
        gate = jnp.sum(probs * onehot.astype(jnp.float32), axis=-1,
                       keepdims=True)

        def accum_chunk(acc, w_ref, origin):
            base = origin * e_loc
            for j in range(e_loc):
                coeff = jnp.where(idx == base + j, gate, 0.0)
                xs = (xv * coeff).astype(jnp.bfloat16)
                acc = acc + jnp.dot(xs, w_ref[j, :, :],
                                    preferred_element_type=jnp.float32)
            return acc

        rdma0 = pltpu.make_async_remote_copy(
            src_ref=ew_bf_ref,
            dst_ref=comm_ref.at[0],
            send_sem=send_sems.at[0],
            recv_sem=recv_sems.at[0],
            device_id=(right,),
            device_id_type=pl.DeviceIdType.MESH,
        )
        rdma0.start()
        acc = jnp.dot(xv_bf, sw_ref[:, :].astype(jnp.bfloat16),
                      preferred_element_type=jnp.float32)
        acc = accum_chunk(acc, ew_bf_ref, my)
        rdma0.wait()

        for hop in range(1, N_DEV - 1):
            rd = pltpu.make_async_remote_copy(
                src_ref=comm_ref.at[hop - 1],
                dst_ref=comm_ref.at[hop],
                send_sem=send_sems.at[hop],
                recv_sem=recv_sems.at[hop],
                device_id=(right,),
                device_id_type=pl.DeviceIdType.MESH,
            )
            rd.start()
            acc = accum_chunk(acc, comm_ref.at[hop - 1],
                              lax.rem(my + N_DEV - hop, N_DEV))
            rd.wait()

        acc = accum_chunk(acc, comm_ref.at[N_DEV - 2],
                          lax.rem(my + 1, N_DEV))
        out_ref[:, :] = acc

    return pl.pallas_call(
        body,
        out_shape=jax.ShapeDtypeStruct((n_tok, h), jnp.float32),
        in_specs=[pl.BlockSpec(memory_space=pltpu.VMEM)] * 5,
        out_specs=pl.BlockSpec(memory_space=pltpu.VMEM),
        scratch_shapes=[
            pltpu.VMEM((N_DEV - 1, e_loc, d, h), jnp.bfloat16),
            pltpu.VMEM((e_loc, d, h), jnp.bfloat16),
            pltpu.SemaphoreType.DMA((N_DEV - 1,)),
            pltpu.SemaphoreType.DMA((N_DEV - 1,)),
        ],
        compiler_params=pltpu.CompilerParams(collective_id=0),
    )(x, router_W, route_idx, expert_W, shared_W)


# device time: 34278 ns/iter; 2.4679x vs baseline; 1.4917x over previous
import jax
import jax.numpy as jnp
from jax import lax
from jax.experimental import pallas as pl
from jax.experimental.pallas import tpu as pltpu

N_DEV = 4


def kernel(x, router_W, route_idx, expert_W, shared_W):
    n_tok, d = x.shape
    n_exp = router_W.shape[1]
    e_loc, _, h = expert_W.shape

    def body(x_ref, rw_ref, idx_ref, ew_ref, sw_ref, out_ref,
             comm_a, comm_b, ew_bf_ref,
             send_a, recv_a, send_b, recv_b):
        my = lax.axis_index("i")
        left = lax.rem(my + N_DEV - 1, N_DEV)
        right = lax.rem(my + 1, N_DEV)

        barrier_sem = pltpu.get_barrier_semaphore()
        for nbr in (left, right):
            pl.semaphore_signal(
                barrier_sem, inc=1,
                device_id=(nbr,), device_id_type=pl.DeviceIdType.MESH,
            )
        pl.semaphore_wait(barrier_sem, 2)

        xv = x_ref[:, :]
        idx = idx_ref[:, :]

        ew_bf_ref[:, :, :] = ew_ref[:, :, :].astype(jnp.bfloat16)
        xv_bf = xv.astype(jnp.bfloat16)

        scores = jnp.dot(xv, rw_ref[:, :], preferred_element_type=jnp.float32)
        s_max = jnp.max(scores, axis=-1, keepdims=True)
        p = jnp.exp(scores - s_max)
        probs = p / jnp.sum(p, axis=-1, keepdims=True)
        onehot = (idx == lax.broadcasted_iota(jnp.int32, (n_tok, n_exp), 1))
        gate = jnp.sum(probs * onehot.astype(jnp.float32), axis=-1,
                       keepdims=True)

        half = e_loc // 2

        def accum(acc, w_ref, origin, off, cnt):
            base = origin * e_loc + off
            for j in range(cnt):
                coeff = jnp.where(idx == base + j, gate, 0.0)
                xs = (xv * coeff).astype(jnp.bfloat16)
                acc = acc + jnp.dot(xs, w_ref[j, :, :],
                                    preferred_element_type=jnp.float32)
            return acc

        def hop_pair(hop, src_a, src_b):
            rd_a = pltpu.make_async_remote_copy(
                src_ref=src_a, dst_ref=comm_a.at[hop],
                send_sem=send_a.at[hop], recv_sem=recv_a.at[hop],
                device_id=(right,), device_id_type=pl.DeviceIdType.MESH,
            )
            rd_b = pltpu.make_async_remote_copy(
                src_ref=src_b, dst_ref=comm_b.at[hop],
                send_sem=send_b.at[hop], recv_sem=recv_b.at[hop],
                device_id=(left,), device_id_type=pl.DeviceIdType.MESH,
            )
            rd_a.start()
            rd_b.start()
            return rd_a, rd_b

        rd_a, rd_b = hop_pair(0, ew_bf_ref.at[0:half], ew_bf_ref.at[half:e_loc])
        acc = jnp.dot(xv_bf, sw_ref[:, :].astype(jnp.bfloat16),
                      preferred_element_type=jnp.float32)
        acc = accum(acc, ew_bf_ref, my, 0, e_loc)
        rd_a.wait()
        rd_b.wait()

        for hop in range(1, N_DEV - 1):
            rd_a, rd_b = hop_pair(hop, comm_a.at[hop - 1], comm_b.at[hop - 1])
            acc = accum(acc, comm_a.at[hop - 1],
                        lax.rem(my + N_DEV - hop, N_DEV), 0, half)
            acc = accum(acc, comm_b.at[hop - 1],
                        lax.rem(my + hop, N_DEV), half, half)
            rd_a.wait()
            rd_b.wait()

        acc = accum(acc, comm_a.at[N_DEV - 2], lax.rem(my + 1, N_DEV), 0, half)
        acc = accum(acc, comm_b.at[N_DEV - 2], lax.rem(my + N_DEV - 1, N_DEV),
                    half, half)
        out_ref[:, :] = acc

    return pl.pallas_call(
        body,
        out_shape=jax.ShapeDtypeStruct((n_tok, h), jnp.float32),
        in_specs=[pl.BlockSpec(memory_space=pltpu.VMEM)] * 5,
        out_specs=pl.BlockSpec(memory_space=pltpu.VMEM),
        scratch_shapes=[
            pltpu.VMEM((N_DEV - 1, e_loc // 2, d, h), jnp.bfloat16),
            pltpu.VMEM((N_DEV - 1, e_loc // 2, d, h), jnp.bfloat16),
            pltpu.VMEM((e_loc, d, h), jnp.bfloat16),
            pltpu.SemaphoreType.DMA((N_DEV - 1,)),
            pltpu.SemaphoreType.DMA((N_DEV - 1,)),
            pltpu.SemaphoreType.DMA((N_DEV - 1,)),
            pltpu.SemaphoreType.DMA((N_DEV - 1,)),
        ],
        compiler_params=pltpu.CompilerParams(collective_id=0),
    )(x, router_W, route_idx, expert_W, shared_W)
